# Initial kernel scaffold; baseline (speedup 1.0000x reference)
import functools

import jax
import jax.numpy as jnp
from jax import lax
from jax.experimental import pallas as pl
from jax.experimental.pallas import tpu as pltpu
from jax.experimental.pallas import tpu_sc as plsc

N = 10000
NP = 10240
E = 320000
D = 128
PD = 64
DEPTH = 16
B = 1024
K = 32
TEMP = 0.5
LAMBDA_1 = 1e-05

BN = 1024
GRID = NP // BN

NW = 32
EW = E // NW
CH = 80
NCH = EW // CH
STRIPE = NP // 16
NZC = STRIPE // CH

GB = 2 * B + B * K
GW = GB // NW
GCH = 64
GNCH = GW // GCH

_HI = lax.Precision.HIGHEST


def _mm_nt(a, b):
    return lax.dot_general(a, b, (((1,), (1,)), ((), ())),
                           preferred_element_type=jnp.float32, precision=_HI)


def _mm_nn(a, b):
    return lax.dot_general(a, b, (((1,), (0,)), ((), ())),
                           preferred_element_type=jnp.float32, precision=_HI)


def _k1_body(es_ref, pos_ref, epw_ref, pwa_ref, pwb_ref, pb_ref, w0_ref,
             b0_ref, x0_ref, h0_ref):
    pids = pos_ref[0, 0, :]
    oh = (pids[:, None] == lax.broadcasted_iota(jnp.int32, (BN, DEPTH), 1))
    ep = _mm_nn(oh.astype(jnp.float32), epw_ref[...])
    x0 = (_mm_nt(es_ref[...], pwa_ref[...]) + _mm_nt(ep, pwb_ref[...])
          + pb_ref[...])
    x0_ref[...] = x0
    h0_ref[...] = jnp.maximum(_mm_nt(x0, w0_ref[...]) + b0_ref[...], 0.0)


def _tc_embed_proj(emb_s_p, pos3d, emb_p_w, proj_Wa, proj_Wb, proj_b2, W0, b02):
    row = lambda i: (i, 0)
    full = lambda i: (0, 0)
    return pl.pallas_call(
        _k1_body,
        grid=(GRID,),
        in_specs=[
            pl.BlockSpec((BN, D), row),
            pl.BlockSpec((1, 1, BN), lambda i: (i, 0, 0)),
            pl.BlockSpec((DEPTH, PD), full),
            pl.BlockSpec((D, D), full),
            pl.BlockSpec((D, PD), full),
            pl.BlockSpec((1, D), full),
            pl.BlockSpec((D, D), full),
            pl.BlockSpec((1, D), full),
        ],
        out_specs=[pl.BlockSpec((BN, D), row), pl.BlockSpec((BN, D), row)],
        out_shape=[jax.ShapeDtypeStruct((NP, D), jnp.float32),
                   jax.ShapeDtypeStruct((NP, D), jnp.float32)],
    )(emb_s_p, pos3d, emb_p_w, proj_Wa, proj_Wb, proj_b2, W0, b02)


def _k2_body(x_ref, ya_ref, yb_ref, w_ref, b_ref, x1_ref, h1_ref):
    x1 = x_ref[...] + ya_ref[...] + yb_ref[...]
    x1_ref[...] = x1
    h1_ref[...] = jnp.maximum(_mm_nt(x1, w_ref[...]) + b_ref[...], 0.0)


def _tc_residual_layer(x, y, W, b2):
    row = lambda i: (i, 0)
    full = lambda i: (0, 0)
    return pl.pallas_call(
        _k2_body,
        grid=(GRID,),
        in_specs=[
            pl.BlockSpec((BN, D), row),
            pl.BlockSpec((BN, D), row),
            pl.BlockSpec((BN, D), lambda i: (GRID + i, 0)),
            pl.BlockSpec((D, D), full),
            pl.BlockSpec((1, D), full),
        ],
        out_specs=[pl.BlockSpec((BN, D), row), pl.BlockSpec((BN, D), row)],
        out_shape=[jax.ShapeDtypeStruct((NP, D), jnp.float32),
                   jax.ShapeDtypeStruct((NP, D), jnp.float32)],
    )(x, y, y, W, b2)


def _k3_body(x_ref, ya_ref, yb_ref, w_ref, b_ref, un_ref):
    x2 = x_ref[...] + ya_ref[...] + yb_ref[...]
    out = _mm_nt(x2, w_ref[...]) + b_ref[...]
    n2 = jnp.sum(out * out, axis=1, keepdims=True)
    na = jnp.maximum(jnp.sqrt(n2), 1e-8)
    un_ref[...] = out / na


def _tc_out_norm(x, y, out_W, out_b2):
    row = lambda i: (i, 0)
    full = lambda i: (0, 0)
    return pl.pallas_call(
        _k3_body,
        grid=(GRID,),
        in_specs=[
            pl.BlockSpec((BN, D), row),
            pl.BlockSpec((BN, D), row),
            pl.BlockSpec((BN, D), lambda i: (GRID + i, 0)),
            pl.BlockSpec((D, D), full),
            pl.BlockSpec((1, D), full),
        ],
        out_specs=pl.BlockSpec((BN, D), row),
        out_shape=jax.ShapeDtypeStruct((NP, D), jnp.float32),
    )(x, y, y, out_W, out_b2)


def _k4_body(g_ref, epw_ref, pw_ref, pb_ref, w0_ref, b0_ref, w1_ref, b1_ref,
             ow_ref, ob_ref, l_ref, lcl_ref, lreg_ref):
    g_s = g_ref[0:B, :]
    g_p = g_ref[B:2 * B, :]
    g_n = g_ref[2 * B:, :].reshape(B, K, D)
    ps = jnp.sum(g_s * g_p, axis=1)
    ns = jnp.sum(g_n * g_s[:, None, :], axis=2)
    eps_ = jnp.exp(ps[:, None] / TEMP)
    ens = jnp.exp(ns / TEMP)
    lc = -jnp.log(eps_ / (eps_ + ens + 1e-08))
    loss_cl = jnp.sum(lc) / (B * K)
    reg = (jnp.sum(epw_ref[...] ** 2) + jnp.sum(pw_ref[...] ** 2)
           + jnp.sum(pb_ref[...] ** 2) + jnp.sum(w0_ref[...] ** 2)
           + jnp.sum(b0_ref[...] ** 2) + jnp.sum(w1_ref[...] ** 2)
           + jnp.sum(b1_ref[...] ** 2) + jnp.sum(ow_ref[...] ** 2)
           + jnp.sum(ob_ref[...] ** 2))
    loss_reg = reg * LAMBDA_1
    lcl_ref[0, 0] = loss_cl
    lreg_ref[0, 0] = loss_reg
    l_ref[0, 0] = loss_cl + loss_reg


def _tc_loss(g_all, emb_p_w, proj_W, proj_b2, W0, b02, W1, b12, out_W, out_b2):
    return pl.pallas_call(
        _k4_body,
        out_shape=[jax.ShapeDtypeStruct((1, 1), jnp.float32)] * 3,
    )(g_all, emb_p_w, proj_W, proj_b2, W0, b02, W1, b12, out_W, out_b2)


_SC_MESH = plsc.VectorSubcoreMesh(core_axis_name="c", subcore_axis_name="s")


def _sc_spmm(h, idxi_r, idxj_r, adj_r):
    @functools.partial(
        pl.kernel,
        out_type=jax.ShapeDtypeStruct((2 * NP, D), jnp.float32),
        mesh=_SC_MESH,
        scratch_types=[
            pltpu.VMEM((NCH, CH), jnp.int32),
            pltpu.VMEM((NCH, CH), jnp.int32),
            pltpu.VMEM((EW,), jnp.float32),
            pltpu.VMEM((CH, D), jnp.float32),
            pltpu.VMEM_SHARED((NP, D), jnp.float32),
            pltpu.SemaphoreType.DMA,
        ],
    )
    def k(h_hbm, ii_hbm, jj_hbm, aa_hbm, out_hbm, ii_v, jj_v, aa_v, rows_v,
          acc_sh, sem):
        c = lax.axis_index("c")
        s = lax.axis_index("s")
        w = s * 2 + c
        pltpu.sync_copy(ii_hbm.at[w], ii_v)
        pltpu.sync_copy(jj_hbm.at[w], jj_v)
        pltpu.sync_copy(aa_hbm.at[w], aa_v)

        z16 = jnp.zeros((16,), jnp.float32)

        def zrow(i, carry):
            for v in range(D // 16):
                rows_v[i, pl.ds(v * 16, 16)] = z16
            return carry

        lax.fori_loop(0, CH, zrow, 0)

        def zcp(i, carry):
            pltpu.sync_copy(rows_v, acc_sh.at[pl.ds(s * STRIPE + i * CH, CH)])
            return carry

        lax.fori_loop(0, NZC, zcp, 0)
        plsc.subcore_barrier()

        def chunk(g, carry):
            pltpu.async_copy(h_hbm.at[jj_v.at[g]], rows_v, sem).wait()

            def scale(e, c2):
                av = plsc.load_gather(
                    aa_v, [jnp.full((16,), g * CH + e, jnp.int32)])
                for v in range(D // 16):
                    sl = pl.ds(v * 16, 16)
                    rows_v[e, sl] = rows_v[e, sl] * av
                return c2

            lax.fori_loop(0, CH, scale, 0)
            pltpu.sync_copy(rows_v, acc_sh.at[ii_v.at[g]], add=True)
            return carry

        lax.fori_loop(0, NCH, chunk, 0)
        plsc.subcore_barrier()

        def drain(i, carry):
            st = s * STRIPE + i * CH
            pltpu.sync_copy(acc_sh.at[pl.ds(st, CH)], rows_v)
            pltpu.sync_copy(rows_v, out_hbm.at[pl.ds(c * NP + st, CH)])
            return carry

        lax.fori_loop(0, NZC, drain, 0)

    return k(h, idxi_r, idxj_r, adj_r)


def _sc_gather(un, idx_r):
    @functools.partial(
        pl.kernel,
        out_type=jax.ShapeDtypeStruct((GB, D), jnp.float32),
        mesh=_SC_MESH,
        scratch_types=[
            pltpu.VMEM((GNCH, GCH), jnp.int32),
            pltpu.VMEM((GCH, D), jnp.float32),
            pltpu.SemaphoreType.DMA,
        ],
    )
    def k(un_hbm, idx_hbm, out_hbm, idx_v, rows_v, sem):
        c = lax.axis_index("c")
        s = lax.axis_index("s")
        w = s * 2 + c
        pltpu.sync_copy(idx_hbm.at[w], idx_v)

        def chunk(g, carry):
            pltpu.async_copy(un_hbm.at[idx_v.at[g]], rows_v, sem).wait()
            pltpu.sync_copy(rows_v,
                            out_hbm.at[pl.ds(w * GW + g * GCH, GCH)])
            return carry

        lax.fori_loop(0, GNCH, chunk, 0)

    return k(un, idx_r)


def kernel(emb_s, edge_index, adj_values, position_ids, sids, pos, negs,
           emb_p_w, proj_W, proj_b, W0, b0, W1, b1, out_W, out_b):
    f32 = jnp.float32
    i32 = jnp.int32

    emb_s_p = jnp.pad(emb_s, ((0, NP - N), (0, 0)))
    pos3d = jnp.pad(position_ids.astype(i32), (0, NP - N)).reshape(GRID, 1, BN)
    proj_Wa = proj_W[:, :D]
    proj_Wb = proj_W[:, D:]
    proj_b2 = proj_b.reshape(1, D)
    b02 = b0.reshape(1, D)
    b12 = b1.reshape(1, D)
    out_b2 = out_b.reshape(1, D)

    idxi_r = edge_index[0].astype(i32).reshape(NW, NCH, CH)
    idxj_r = edge_index[1].astype(i32).reshape(NW, NCH, CH)
    adj_r = adj_values.astype(f32).reshape(NW, EW)

    all_idx = jnp.concatenate(
        [sids.astype(i32), pos.astype(i32),
         jnp.swapaxes(negs, 0, 1).reshape(-1).astype(i32)]
    ).reshape(NW, GNCH, GCH)

    x0, h0 = _tc_embed_proj(emb_s_p, pos3d, emb_p_w, proj_Wa, proj_Wb,
                            proj_b2, W0, b02)
    y0 = _sc_spmm(h0, idxi_r, idxj_r, adj_r)
    x1, h1 = _tc_residual_layer(x0, y0, W1, b12)
    y1 = _sc_spmm(h1, idxi_r, idxj_r, adj_r)
    un = _tc_out_norm(x1, y1, out_W, out_b2)
    g_all = _sc_gather(un, all_idx)
    loss, loss_cl, loss_reg = _tc_loss(g_all, emb_p_w, proj_W, proj_b2, W0,
                                       b02, W1, b12, out_W, out_b2)
    return (loss[0, 0], loss_cl[0, 0], loss_reg[0, 0])


# speedup vs baseline: 5.2846x; 5.2846x over previous
import functools

import jax
import jax.numpy as jnp
from jax import lax
from jax.experimental import pallas as pl
from jax.experimental.pallas import tpu as pltpu
from jax.experimental.pallas import tpu_sc as plsc

N = 10000
NP = 10240
E = 320000
D = 128
PD = 64
DEPTH = 16
B = 1024
K = 32
TEMP = 0.5
LAMBDA_1 = 1e-05

BN = 1024
GRID = NP // BN

NW = 32
EW = E // NW
CH = 80
NCH = EW // CH
CB = 25
NB = NCH // CB
STRIPE = NP // 16
NZC = STRIPE // CH

GB = 2 * B + B * K
GW = GB // NW
GCH = 64
GNCH = GW // GCH

_HI = lax.Precision.HIGHEST


def _mm_nt(a, b):
    return lax.dot_general(a, b, (((1,), (1,)), ((), ())),
                           preferred_element_type=jnp.float32, precision=_HI)


def _mm_nn(a, b):
    return lax.dot_general(a, b, (((1,), (0,)), ((), ())),
                           preferred_element_type=jnp.float32, precision=_HI)


def _k1_body(es_ref, pos_ref, epw_ref, pwa_ref, pwb_ref, pb_ref, w0_ref,
             b0_ref, x0_ref, h0_ref):
    pids = pos_ref[0, 0, :]
    oh = (pids[:, None] == lax.broadcasted_iota(jnp.int32, (BN, DEPTH), 1))
    ep = _mm_nn(oh.astype(jnp.float32), epw_ref[...])
    x0 = (_mm_nt(es_ref[...], pwa_ref[...]) + _mm_nt(ep, pwb_ref[...])
          + pb_ref[...])
    x0_ref[...] = x0
    h0_ref[...] = jnp.maximum(_mm_nt(x0, w0_ref[...]) + b0_ref[...], 0.0)


def _tc_embed_proj(emb_s_p, pos3d, emb_p_w, proj_Wa, proj_Wb, proj_b2, W0, b02):
    row = lambda i: (i, 0)
    full = lambda i: (0, 0)
    return pl.pallas_call(
        _k1_body,
        grid=(GRID,),
        in_specs=[
            pl.BlockSpec((BN, D), row),
            pl.BlockSpec((1, 1, BN), lambda i: (i, 0, 0)),
            pl.BlockSpec((DEPTH, PD), full),
            pl.BlockSpec((D, D), full),
            pl.BlockSpec((D, PD), full),
            pl.BlockSpec((1, D), full),
            pl.BlockSpec((D, D), full),
            pl.BlockSpec((1, D), full),
        ],
        out_specs=[pl.BlockSpec((BN, D), row), pl.BlockSpec((BN, D), row)],
        out_shape=[jax.ShapeDtypeStruct((NP, D), jnp.float32),
                   jax.ShapeDtypeStruct((NP, D), jnp.float32)],
    )(emb_s_p, pos3d, emb_p_w, proj_Wa, proj_Wb, proj_b2, W0, b02)


def _k2_body(x_ref, ya_ref, yb_ref, w_ref, b_ref, x1_ref, h1_ref):
    x1 = x_ref[...] + ya_ref[...] + yb_ref[...]
    x1_ref[...] = x1
    h1_ref[...] = jnp.maximum(_mm_nt(x1, w_ref[...]) + b_ref[...], 0.0)


def _tc_residual_layer(x, y, W, b2):
    row = lambda i: (i, 0)
    full = lambda i: (0, 0)
    return pl.pallas_call(
        _k2_body,
        grid=(GRID,),
        in_specs=[
            pl.BlockSpec((BN, D), row),
            pl.BlockSpec((BN, D), row),
            pl.BlockSpec((BN, D), lambda i: (GRID + i, 0)),
            pl.BlockSpec((D, D), full),
            pl.BlockSpec((1, D), full),
        ],
        out_specs=[pl.BlockSpec((BN, D), row), pl.BlockSpec((BN, D), row)],
        out_shape=[jax.ShapeDtypeStruct((NP, D), jnp.float32),
                   jax.ShapeDtypeStruct((NP, D), jnp.float32)],
    )(x, y, y, W, b2)


def _k3_body(x_ref, ya_ref, yb_ref, w_ref, b_ref, un_ref):
    x2 = x_ref[...] + ya_ref[...] + yb_ref[...]
    out = _mm_nt(x2, w_ref[...]) + b_ref[...]
    n2 = jnp.sum(out * out, axis=1, keepdims=True)
    na = jnp.maximum(jnp.sqrt(n2), 1e-8)
    un_ref[...] = out / na


def _tc_out_norm(x, y, out_W, out_b2):
    row = lambda i: (i, 0)
    full = lambda i: (0, 0)
    return pl.pallas_call(
        _k3_body,
        grid=(GRID,),
        in_specs=[
            pl.BlockSpec((BN, D), row),
            pl.BlockSpec((BN, D), row),
            pl.BlockSpec((BN, D), lambda i: (GRID + i, 0)),
            pl.BlockSpec((D, D), full),
            pl.BlockSpec((1, D), full),
        ],
        out_specs=pl.BlockSpec((BN, D), row),
        out_shape=jax.ShapeDtypeStruct((NP, D), jnp.float32),
    )(x, y, y, out_W, out_b2)


def _k4_body(g_ref, epw_ref, pw_ref, pb_ref, w0_ref, b0_ref, w1_ref, b1_ref,
             ow_ref, ob_ref, l_ref, lcl_ref, lreg_ref):
    g_s = g_ref[0:B, :]
    g_p = g_ref[B:2 * B, :]
    g_n = g_ref[2 * B:, :].reshape(B, K, D)
    ps = jnp.sum(g_s * g_p, axis=1)
    ns = jnp.sum(g_n * g_s[:, None, :], axis=2)
    eps_ = jnp.exp(ps[:, None] / TEMP)
    ens = jnp.exp(ns / TEMP)
    lc = -jnp.log(eps_ / (eps_ + ens + 1e-08))
    loss_cl = jnp.sum(lc) / (B * K)
    reg = (jnp.sum(epw_ref[...] ** 2) + jnp.sum(pw_ref[...] ** 2)
           + jnp.sum(pb_ref[...] ** 2) + jnp.sum(w0_ref[...] ** 2)
           + jnp.sum(b0_ref[...] ** 2) + jnp.sum(w1_ref[...] ** 2)
           + jnp.sum(b1_ref[...] ** 2) + jnp.sum(ow_ref[...] ** 2)
           + jnp.sum(ob_ref[...] ** 2))
    loss_reg = reg * LAMBDA_1
    lcl_ref[...] = jnp.reshape(loss_cl, (1, 1))
    lreg_ref[...] = jnp.reshape(loss_reg, (1, 1))
    l_ref[...] = jnp.reshape(loss_cl + loss_reg, (1, 1))


def _tc_loss(g_all, emb_p_w, proj_W, proj_b2, W0, b02, W1, b12, out_W, out_b2):
    return pl.pallas_call(
        _k4_body,
        out_shape=[jax.ShapeDtypeStruct((1, 1), jnp.float32)] * 3,
    )(g_all, emb_p_w, proj_W, proj_b2, W0, b02, W1, b12, out_W, out_b2)


@functools.cache
def _sc_mesh():
    return plsc.VectorSubcoreMesh(core_axis_name="c", subcore_axis_name="s")


def _sc_spmm(h, idxi_r, idxj_r, adj_r):
    @functools.partial(
        pl.kernel,
        out_type=jax.ShapeDtypeStruct((2 * NP, D), jnp.float32),
        mesh=_sc_mesh(),
        scratch_types=[
            pltpu.VMEM((CB, CH), jnp.int32),
            pltpu.VMEM((CB, CH), jnp.int32),
            pltpu.VMEM((CB, CH), jnp.float32),
            pltpu.VMEM((CH, D), jnp.float32),
            pltpu.VMEM_SHARED((NP, D), jnp.float32),
            pltpu.SemaphoreType.DMA,
        ],
    )
    def k(h_hbm, ii_hbm, jj_hbm, aa_hbm, out_hbm, ii_v, jj_v, aa_v, rows_v,
          acc_sh, sem):
        c = lax.axis_index("c")
        s = lax.axis_index("s")
        w = s * 2 + c

        z16 = jnp.zeros((16,), jnp.float32)

        def zrow(i, carry):
            for v in range(D // 16):
                rows_v[i, pl.ds(v * 16, 16)] = z16
            return carry

        lax.fori_loop(0, CH, zrow, 0)

        def zcp(i, carry):
            pltpu.sync_copy(rows_v, acc_sh.at[pl.ds(s * STRIPE + i * CH, CH)])
            return carry

        lax.fori_loop(0, NZC, zcp, 0)
        plsc.subcore_barrier()

        def block(blk, carry0):
            pltpu.sync_copy(ii_hbm.at[w, blk], ii_v)
            pltpu.sync_copy(jj_hbm.at[w, blk], jj_v)
            pltpu.sync_copy(aa_hbm.at[w, blk], aa_v)

            def chunk(g, carry):
                pltpu.async_copy(h_hbm.at[jj_v.at[g]], rows_v, sem).wait()

                def scale(g2, c2):
                    a16 = aa_v[g, pl.ds(g2 * 16, 16)]
                    erow = g2 * 16
                    for e16 in range(16):
                        av = a16.at[jnp.full((16,), e16, jnp.int32)].get(
                            mode="promise_in_bounds")
                        for v in range(D // 16):
                            sl = pl.ds(v * 16, 16)
                            rows_v[erow + e16, sl] = (
                                rows_v[erow + e16, sl] * av)
                    return c2

                lax.fori_loop(0, CH // 16, scale, 0)
                pltpu.sync_copy(rows_v, acc_sh.at[ii_v.at[g]], add=True)
                return carry

            lax.fori_loop(0, CB, chunk, 0)
            return carry0

        lax.fori_loop(0, NB, block, 0)
        plsc.subcore_barrier()

        def drain(i, carry):
            st = s * STRIPE + i * CH
            pltpu.sync_copy(acc_sh.at[pl.ds(st, CH)], rows_v)
            pltpu.sync_copy(rows_v, out_hbm.at[pl.ds(c * NP + st, CH)])
            return carry

        lax.fori_loop(0, NZC, drain, 0)

    return k(h, idxi_r, idxj_r, adj_r)


def _sc_gather(un, idx_r):
    @functools.partial(
        pl.kernel,
        out_type=jax.ShapeDtypeStruct((GB, D), jnp.float32),
        mesh=_sc_mesh(),
        scratch_types=[
            pltpu.VMEM((GNCH, GCH), jnp.int32),
            pltpu.VMEM((GCH, D), jnp.float32),
            pltpu.SemaphoreType.DMA,
        ],
    )
    def k(un_hbm, idx_hbm, out_hbm, idx_v, rows_v, sem):
        c = lax.axis_index("c")
        s = lax.axis_index("s")
        w = s * 2 + c
        pltpu.sync_copy(idx_hbm.at[w], idx_v)

        def chunk(g, carry):
            pltpu.async_copy(un_hbm.at[idx_v.at[g]], rows_v, sem).wait()
            pltpu.sync_copy(rows_v,
                            out_hbm.at[pl.ds(w * GW + g * GCH, GCH)])
            return carry

        lax.fori_loop(0, GNCH, chunk, 0)

    return k(un, idx_r)


def kernel(emb_s, edge_index, adj_values, position_ids, sids, pos, negs,
           emb_p_w, proj_W, proj_b, W0, b0, W1, b1, out_W, out_b):
    f32 = jnp.float32
    i32 = jnp.int32

    emb_s_p = jnp.pad(emb_s, ((0, NP - N), (0, 0)))
    pos3d = jnp.pad(position_ids.astype(i32), (0, NP - N)).reshape(GRID, 1, BN)
    proj_Wa = proj_W[:, :D]
    proj_Wb = proj_W[:, D:]
    proj_b2 = proj_b.reshape(1, D)
    b02 = b0.reshape(1, D)
    b12 = b1.reshape(1, D)
    out_b2 = out_b.reshape(1, D)

    idxi_r = edge_index[0].astype(i32).reshape(NW, NB, CB, CH)
    idxj_r = edge_index[1].astype(i32).reshape(NW, NB, CB, CH)
    adj_r = adj_values.astype(f32).reshape(NW, NB, CB, CH)

    all_idx = jnp.concatenate(
        [sids.astype(i32), pos.astype(i32),
         jnp.swapaxes(negs, 0, 1).reshape(-1).astype(i32)]
    ).reshape(NW, GNCH, GCH)

    x0, h0 = _tc_embed_proj(emb_s_p, pos3d, emb_p_w, proj_Wa, proj_Wb,
                            proj_b2, W0, b02)
    y0 = _sc_spmm(h0, idxi_r, idxj_r, adj_r)
    x1, h1 = _tc_residual_layer(x0, y0, W1, b12)
    y1 = _sc_spmm(h1, idxi_r, idxj_r, adj_r)
    un = _tc_out_norm(x1, y1, out_W, out_b2)
    g_all = _sc_gather(un, all_idx)
    loss, loss_cl, loss_reg = _tc_loss(g_all, emb_p_w, proj_W, proj_b2, W0,
                                       b02, W1, b12, out_W, out_b2)
    return (loss[0, 0], loss_cl[0, 0], loss_reg[0, 0])
